# baseline (speedup 1.0000x reference)
import functools

import jax
import jax.numpy as jnp
from jax import lax
from jax.experimental import pallas as pl
from jax.experimental.pallas import tpu as pltpu
from jax.experimental.pallas import tpu_sc as plsc

EMBED_D = 64
NUM_CORES = 2
NUM_SUBCORES = 16
NUM_WORKERS = NUM_CORES * NUM_SUBCORES
CHUNK = 128
NBUF = 4


def _make_lookup(total_rows: int):
  packed_rows = total_rows // 2
  rows_per_w = packed_rows // NUM_WORKERS
  n_groups = rows_per_w // CHUNK
  idx_rows_per_w = n_groups

  mesh = plsc.VectorSubcoreMesh(
      core_axis_name="c", subcore_axis_name="s", num_cores=NUM_CORES)

  @functools.partial(
      pl.kernel,
      out_type=jax.ShapeDtypeStruct((packed_rows, 2 * EMBED_D), jnp.float32),
      mesh=mesh,
      compiler_params=pltpu.CompilerParams(use_tc_tiling_on_sc=False),
      scratch_types=[
          pltpu.VMEM((2 * idx_rows_per_w, CHUNK), jnp.int32),
          pltpu.VMEM((CHUNK, EMBED_D), jnp.float32),
          pltpu.VMEM((CHUNK, EMBED_D), jnp.float32),
          pltpu.VMEM((CHUNK, EMBED_D), jnp.float32),
          pltpu.VMEM((CHUNK, EMBED_D), jnp.float32),
          pltpu.VMEM((CHUNK, EMBED_D), jnp.float32),
          pltpu.VMEM((CHUNK, EMBED_D), jnp.float32),
          pltpu.VMEM((CHUNK, EMBED_D), jnp.float32),
          pltpu.VMEM((CHUNK, EMBED_D), jnp.float32),
          pltpu.SemaphoreType.DMA,
          pltpu.SemaphoreType.DMA,
          pltpu.SemaphoreType.DMA,
          pltpu.SemaphoreType.DMA,
          pltpu.SemaphoreType.DMA,
          pltpu.SemaphoreType.DMA,
          pltpu.SemaphoreType.DMA,
          pltpu.SemaphoreType.DMA,
      ],
  )
  def lookup(table_hbm, idx_even_hbm, idx_odd_hbm, out_hbm, idx_v,
             e0, o0, e1, o1, e2, o2, e3, o3,
             g0, g1, g2, g3, s0, s1, s2, s3):
    wid = lax.axis_index("s") * NUM_CORES + lax.axis_index("c")
    idx_base = wid * idx_rows_per_w
    out_base = wid * rows_per_w

    pltpu.sync_copy(idx_even_hbm.at[pl.ds(idx_base, idx_rows_per_w)],
                    idx_v.at[pl.ds(0, idx_rows_per_w)])
    pltpu.sync_copy(idx_odd_hbm.at[pl.ds(idx_base, idx_rows_per_w)],
                    idx_v.at[pl.ds(idx_rows_per_w, idx_rows_per_w)])

    bufs = ((e0, o0, g0, s0), (e1, o1, g1, s1), (e2, o2, g2, s2),
            (e3, o3, g3, s3))

    def fire_gathers(g):
      be, bo, gsem, _ = bufs[g % NBUF]
      pltpu.async_copy(table_hbm.at[idx_v.at[g]], be, gsem)
      pltpu.async_copy(table_hbm.at[idx_v.at[idx_rows_per_w + g]], bo, gsem)

    def wait_gathers(g):
      be, bo, gsem, _ = bufs[g % NBUF]
      pltpu.make_async_copy(table_hbm.at[idx_v.at[g]], be, gsem).wait()
      pltpu.make_async_copy(table_hbm.at[idx_v.at[idx_rows_per_w + g]], bo,
                            gsem).wait()

    def out_slices(g):
      rows = out_hbm.at[pl.ds(out_base + g * CHUNK, CHUNK)]
      return rows.at[:, pl.ds(0, EMBED_D)], rows.at[:, pl.ds(EMBED_D, EMBED_D)]

    def fire_store(g):
      be, bo, _, ssem = bufs[g % NBUF]
      de, do = out_slices(g)
      pltpu.async_copy(be, de, ssem)
      pltpu.async_copy(bo, do, ssem)

    def wait_store(g):
      be, bo, _, ssem = bufs[g % NBUF]
      de, do = out_slices(g)
      pltpu.make_async_copy(be, de, ssem).wait()
      pltpu.make_async_copy(bo, do, ssem).wait()

    for g in range(min(NBUF - 1, n_groups)):
      fire_gathers(g)
    for g in range(n_groups):
      nxt = g + NBUF - 1
      if nxt < n_groups:
        if g >= 1:
          wait_store(g - 1)
        fire_gathers(nxt)
      wait_gathers(g)
      fire_store(g)

    for g in range(max(0, n_groups - NBUF + 1), n_groups):
      wait_store(g)

  return lookup


def _tc_transpose(x, batch: int, seq: int):
  bblk = 128
  n_bt = batch // bblk
  rows_per_bt = bblk * seq // 2

  def body(x_ref, o_ref):
    xb = x_ref[...]
    x3 = xb.reshape(bblk, seq // 2, 2 * EMBED_D)
    z = jnp.transpose(x3, (1, 2, 0))
    o_ref[...] = z.reshape(seq, EMBED_D, bblk)

  return pl.pallas_call(
      body,
      grid=(n_bt,),
      in_specs=[pl.BlockSpec((rows_per_bt, 2 * EMBED_D), lambda bt: (bt, 0))],
      out_specs=pl.BlockSpec((seq, EMBED_D, bblk), lambda bt: (0, 0, bt)),
      out_shape=jax.ShapeDtypeStruct((seq, EMBED_D, batch), jnp.float32),
  )(x)


def kernel(ids, W):
  batch, seq = ids.shape
  flat_ids = ids.reshape(-1).astype(jnp.int32)
  total_rows = flat_ids.shape[0]
  idx_even = flat_ids[0::2].reshape(total_rows // (2 * CHUNK), CHUNK)
  idx_odd = flat_ids[1::2].reshape(total_rows // (2 * CHUNK), CHUNK)
  packed = _make_lookup(total_rows)(W, idx_even, idx_odd)
  o3 = _tc_transpose(packed, batch, seq)
  return jnp.transpose(o3, (2, 0, 1))


# speedup vs baseline: 1.1585x; 1.1585x over previous
import functools

import jax
import jax.numpy as jnp
from jax import lax
from jax.experimental import pallas as pl
from jax.experimental.pallas import tpu as pltpu
from jax.experimental.pallas import tpu_sc as plsc

EMBED_D = 64
NUM_CORES = 2
NUM_SUBCORES = 16
NUM_WORKERS = NUM_CORES * NUM_SUBCORES
BBLK = 128
NBUF = 4


def _make_lookup(batch: int, seq: int):
  assert batch // NUM_WORKERS == BBLK and seq % 2 == 0
  n_pairs = seq // 2

  mesh = plsc.VectorSubcoreMesh(
      core_axis_name="c", subcore_axis_name="s", num_cores=NUM_CORES)

  @functools.partial(
      pl.kernel,
      out_type=jax.ShapeDtypeStruct((n_pairs, batch, 2 * EMBED_D),
                                    jnp.float32),
      mesh=mesh,
      compiler_params=pltpu.CompilerParams(use_tc_tiling_on_sc=False),
      scratch_types=[
          pltpu.VMEM((seq, BBLK), jnp.int32),
          pltpu.VMEM((BBLK, EMBED_D), jnp.float32),
          pltpu.VMEM((BBLK, EMBED_D), jnp.float32),
          pltpu.VMEM((BBLK, EMBED_D), jnp.float32),
          pltpu.VMEM((BBLK, EMBED_D), jnp.float32),
          pltpu.VMEM((BBLK, EMBED_D), jnp.float32),
          pltpu.VMEM((BBLK, EMBED_D), jnp.float32),
          pltpu.VMEM((BBLK, EMBED_D), jnp.float32),
          pltpu.VMEM((BBLK, EMBED_D), jnp.float32),
          pltpu.SemaphoreType.DMA,
          pltpu.SemaphoreType.DMA,
          pltpu.SemaphoreType.DMA,
          pltpu.SemaphoreType.DMA,
          pltpu.SemaphoreType.DMA,
          pltpu.SemaphoreType.DMA,
          pltpu.SemaphoreType.DMA,
          pltpu.SemaphoreType.DMA,
      ],
  )
  def lookup(table_hbm, idst_hbm, out_hbm, idx_v,
             e0, o0, e1, o1, e2, o2, e3, o3,
             g0, g1, g2, g3, s0, s1, s2, s3):
    wid = lax.axis_index("s") * NUM_CORES + lax.axis_index("c")
    b0 = wid * BBLK

    pltpu.sync_copy(idst_hbm.at[:, pl.ds(b0, BBLK)], idx_v)

    bufs = ((e0, o0, g0, s0), (e1, o1, g1, s1), (e2, o2, g2, s2),
            (e3, o3, g3, s3))

    def fire_gathers(j):
      be, bo, gsem, _ = bufs[j % NBUF]
      pltpu.async_copy(table_hbm.at[idx_v.at[2 * j]], be, gsem)
      pltpu.async_copy(table_hbm.at[idx_v.at[2 * j + 1]], bo, gsem)

    def wait_gathers(j):
      be, bo, gsem, _ = bufs[j % NBUF]
      pltpu.make_async_copy(table_hbm.at[idx_v.at[2 * j]], be, gsem).wait()
      pltpu.make_async_copy(table_hbm.at[idx_v.at[2 * j + 1]], bo,
                            gsem).wait()

    def out_slices(j):
      rows = out_hbm.at[j].at[pl.ds(b0, BBLK)]
      return rows.at[:, pl.ds(0, EMBED_D)], rows.at[:, pl.ds(EMBED_D,
                                                             EMBED_D)]

    def fire_store(j):
      be, bo, _, ssem = bufs[j % NBUF]
      de, do = out_slices(j)
      pltpu.async_copy(be, de, ssem)
      pltpu.async_copy(bo, do, ssem)

    def wait_store(j):
      be, bo, _, ssem = bufs[j % NBUF]
      de, do = out_slices(j)
      pltpu.make_async_copy(be, de, ssem).wait()
      pltpu.make_async_copy(bo, do, ssem).wait()

    for j in range(min(NBUF - 1, n_pairs)):
      fire_gathers(j)
    for j in range(n_pairs):
      nxt = j + NBUF - 1
      if nxt < n_pairs:
        if j >= 1:
          wait_store(j - 1)
        fire_gathers(nxt)
      wait_gathers(j)
      fire_store(j)

    for j in range(max(0, n_pairs - NBUF + 1), n_pairs):
      wait_store(j)

  return lookup


def _tc_transpose(x, batch: int, seq: int):
  n_pairs = seq // 2
  bq = 512

  def body(x_ref, o_ref):
    xb = x_ref[0]
    o_ref[...] = xb.T.reshape(2, EMBED_D, bq)

  return pl.pallas_call(
      body,
      grid=(n_pairs, batch // bq),
      in_specs=[pl.BlockSpec((1, bq, 2 * EMBED_D), lambda j, b: (j, b, 0))],
      out_specs=pl.BlockSpec((2, EMBED_D, bq), lambda j, b: (j, 0, b)),
      out_shape=jax.ShapeDtypeStruct((seq, EMBED_D, batch), jnp.float32),
  )(x)


def kernel(ids, W):
  batch, seq = ids.shape
  idst = ids.T.astype(jnp.int32)
  slabs = _make_lookup(batch, seq)(W, idst)
  o3 = _tc_transpose(slabs, batch, seq)
  return jnp.transpose(o3, (2, 0, 1))
